# baseline (speedup 1.0000x reference)
import functools

import jax
import jax.numpy as jnp
from jax import lax
from jax.experimental import pallas as pl
from jax.experimental.pallas import tpu as pltpu
from jax.experimental.pallas import tpu_sc as plsc

NC = 2
NS = 16
NW = NC * NS
LANES = 16

BATCH = 16384
EMBED_DIM = 64
B_PER_W = BATCH // NW
CHUNK = 128
N_CHUNKS = B_PER_W // CHUNK
N_GROUPS = B_PER_W // LANES


def _mf_body(user_ref, item_ref, uemb_ref, iemb_ref, gb_ref, ubias_ref,
             ibias_ref, out_ref,
             idx_u, idx_i, u_rows, i_rows, ub_v, ib_v, gb_v, out_v, sem):
    wid = lax.axis_index("s") * NC + lax.axis_index("c")

    pltpu.sync_copy(user_ref.at[wid], idx_u)
    pltpu.sync_copy(item_ref.at[wid], idx_i)
    pltpu.sync_copy(gb_ref, gb_v)

    copies = []
    for c in range(N_CHUNKS):
        iu = idx_u.at[c]
        ii = idx_i.at[c]
        sl = pl.ds(c * CHUNK, CHUNK)
        copies.append(pltpu.async_copy(uemb_ref.at[iu], u_rows.at[sl], sem))
        copies.append(pltpu.async_copy(iemb_ref.at[ii], i_rows.at[sl], sem))
        copies.append(pltpu.async_copy(ubias_ref.at[iu], ub_v.at[sl], sem))
        copies.append(pltpu.async_copy(ibias_ref.at[ii], ib_v.at[sl], sem))
    for cp in copies:
        cp.wait()

    lane = lax.iota(jnp.int32, LANES)
    gb_vec = gb_v[...]

    def group_body(g, _):
        base = g * LANES
        row = base + lane

        def dot_step(j, acc):
            col = (j & 48) + ((lane + j) & 15)
            u = plsc.load_gather(u_rows, [row, col])
            v = plsc.load_gather(i_rows, [row, col])
            return acc + u * v

        acc0 = gb_vec + ub_v[pl.ds(base, LANES)] + ib_v[pl.ds(base, LANES)]
        acc = lax.fori_loop(0, EMBED_DIM, dot_step, acc0)
        out_v[pl.ds(base, LANES)] = acc
        return 0

    lax.fori_loop(0, N_GROUPS, group_body, 0)

    pltpu.sync_copy(out_v, out_ref.at[wid])


@functools.partial(jax.jit, static_argnames=())
def _mf_sc(user3, item3, user_emb, item_emb, gb16, user_bias, item_bias):
    mesh = plsc.VectorSubcoreMesh(core_axis_name="c", subcore_axis_name="s")
    k = pl.kernel(
        _mf_body,
        out_type=jax.ShapeDtypeStruct((NW, B_PER_W), jnp.float32),
        mesh=mesh,
        compiler_params=pltpu.CompilerParams(
            needs_layout_passes=False, use_tc_tiling_on_sc=False),
        scratch_types=[
            pltpu.VMEM((N_CHUNKS, CHUNK), jnp.int32),
            pltpu.VMEM((N_CHUNKS, CHUNK), jnp.int32),
            pltpu.VMEM((B_PER_W, EMBED_DIM), jnp.float32),
            pltpu.VMEM((B_PER_W, EMBED_DIM), jnp.float32),
            pltpu.VMEM((B_PER_W,), jnp.float32),
            pltpu.VMEM((B_PER_W,), jnp.float32),
            pltpu.VMEM((LANES,), jnp.float32),
            pltpu.VMEM((B_PER_W,), jnp.float32),
            pltpu.SemaphoreType.DMA,
        ],
    )
    return k(user3, item3, user_emb, item_emb, gb16, user_bias, item_bias)


def kernel(user, item, user_emb, item_emb, global_bias, user_bias, item_bias):
    user3 = user.reshape(NW, N_CHUNKS, CHUNK)
    item3 = item.reshape(NW, N_CHUNKS, CHUNK)
    gb16 = jnp.broadcast_to(global_bias, (LANES,))
    out = _mf_sc(user3, item3, user_emb, item_emb, gb16, user_bias, item_bias)
    return out.reshape(BATCH)
